# Initial kernel scaffold; baseline (speedup 1.0000x reference)
import jax
import jax.numpy as jnp
from jax.experimental import pallas as pl


def kernel(edge_index, knowledge_tag, test_id, big_category, day_diff, edge_weight, user_emb, item_emb, tag_emb, testid_emb, bigcat_emb, daydiff_emb):
    raise NotImplementedError("write your pallas kernel here")


# speedup vs baseline: 1.9367x; 1.9367x over previous
import jax
import jax.numpy as jnp
from jax import lax
from jax.experimental import pallas as pl
from jax.experimental.pallas import tpu as pltpu
from jax.experimental.pallas import tpu_sc as plsc

N_USER = 25000
N_ITEM = 25000
N = N_USER + N_ITEM
D = 64
E = 800000
NC, NS, L = 2, 16, 16
NW = NC * NS
BLK = 128

A_BLOCKS = (N_USER + BLK - 1) // BLK
A_PER_W = (A_BLOCKS + NW - 1) // NW
A_LAST_START = A_BLOCKS - A_PER_W
A_ROW_LAST = N_USER - BLK

E_BLOCKS = E // BLK
B_PER_W = (E_BLOCKS + NW - 1) // NW
B_LAST_START = E_BLOCKS - B_PER_W

S_USER = 1.0 / 6.0
S_ITEM = 1.0 / 12.0


def _worker_id():
    return lax.axis_index("s") * NC + lax.axis_index("c")


def _build_x_body(user_emb, item_emb, day_diff, ktag, tid, bcat,
                  dd_emb, tag_emb, tid_emb, bc_emb,
                  x_out, rows_v, idx_v, sem):
    wid = _worker_id()
    jstart = jnp.minimum(wid * A_PER_W, A_LAST_START)
    for u in range(A_PER_W):
        blk = jstart + u
        rs = pl.multiple_of(jnp.minimum(blk * BLK, A_ROW_LAST), 8)
        pltpu.sync_copy(user_emb.at[pl.ds(rs, BLK)], rows_v)
        pltpu.sync_copy(day_diff.at[pl.ds(rs, BLK)], idx_v)
        pltpu.async_copy(dd_emb.at[idx_v], rows_v, sem, add=True).wait()
        pltpu.sync_copy(rows_v, x_out.at[pl.ds(rs, BLK)])
        pltpu.sync_copy(item_emb.at[pl.ds(rs, BLK)], rows_v)
        for idx_hbm, tab in ((ktag, tag_emb), (tid, tid_emb), (bcat, bc_emb)):
            pltpu.sync_copy(idx_hbm.at[pl.ds(rs, BLK)], idx_v)
            pltpu.async_copy(tab.at[idx_v], rows_v, sem, add=True).wait()
        pltpu.sync_copy(rows_v, x_out.at[pl.ds(N_USER + rs, BLK)])


def _edge_dot_body(x_hbm, src_hbm, dst_hbm, out_hbm,
                   sidx, didx, srow, drow, outv, sem_s, sem_d):
    wid = _worker_id()
    bstart = jnp.minimum(wid * B_PER_W, B_LAST_START)
    pltpu.sync_copy(src_hbm.at[pl.ds(bstart, B_PER_W)], sidx)
    pltpu.sync_copy(dst_hbm.at[pl.ds(bstart, B_PER_W)], didx)

    def step(t, carry):
        cs = pltpu.async_copy(x_hbm.at[sidx.at[t]], srow, sem_s)
        cd = pltpu.async_copy(x_hbm.at[didx.at[t]], drow, sem_d)
        cs.wait()
        cd.wait()
        for g in range(BLK // L):
            rowid = lax.iota(jnp.int32, L) + (g * L)
            si = sidx[t, pl.ds(g * L, L)]
            di = didx[t, pl.ds(g * L, L)]
            f = (jnp.where(si < N_USER, jnp.float32(S_USER), jnp.float32(S_ITEM))
                 * jnp.where(di < N_USER, jnp.float32(S_USER), jnp.float32(S_ITEM)))
            accs = [jnp.zeros((L,), jnp.float32) for _ in range(4)]
            for d in range(D):
                col = jnp.full((L,), d, jnp.int32)
                va = plsc.load_gather(srow, [rowid, col])
                vb = plsc.load_gather(drow, [rowid, col])
                accs[d % 4] = accs[d % 4] + va * vb
            acc = (accs[0] + accs[1]) + (accs[2] + accs[3])
            outv[t, pl.ds(g * L, L)] = acc * f
        return carry

    lax.fori_loop(0, B_PER_W, step, 0)
    pltpu.sync_copy(outv, out_hbm.at[pl.ds(bstart, B_PER_W)])


def kernel(edge_index, knowledge_tag, test_id, big_category, day_diff,
           edge_weight, user_emb, item_emb, tag_emb, testid_emb,
           bigcat_emb, daydiff_emb):
    del edge_weight
    src2d = edge_index[0].reshape(E_BLOCKS, BLK)
    dst2d = edge_index[1].reshape(E_BLOCKS, BLK)
    mesh = plsc.VectorSubcoreMesh(core_axis_name="c", subcore_axis_name="s")
    params = pltpu.CompilerParams(use_tc_tiling_on_sc=False,
                                  needs_layout_passes=False)

    x = pl.kernel(
        _build_x_body,
        out_type=jax.ShapeDtypeStruct((N, D), jnp.float32),
        mesh=mesh,
        scratch_types=[
            pltpu.VMEM((BLK, D), jnp.float32),
            pltpu.VMEM((BLK,), jnp.int32),
            pltpu.SemaphoreType.DMA,
        ],
        compiler_params=params,
    )(user_emb, item_emb, day_diff, knowledge_tag, test_id, big_category,
      daydiff_emb, tag_emb, testid_emb, bigcat_emb)

    out2d = pl.kernel(
        _edge_dot_body,
        out_type=jax.ShapeDtypeStruct((E_BLOCKS, BLK), jnp.float32),
        mesh=mesh,
        scratch_types=[
            pltpu.VMEM((B_PER_W, BLK), jnp.int32),
            pltpu.VMEM((B_PER_W, BLK), jnp.int32),
            pltpu.VMEM((BLK, D), jnp.float32),
            pltpu.VMEM((BLK, D), jnp.float32),
            pltpu.VMEM((B_PER_W, BLK), jnp.float32),
            pltpu.SemaphoreType.DMA,
            pltpu.SemaphoreType.DMA,
        ],
        compiler_params=params,
    )(x, src2d, dst2d)

    return out2d.reshape(E)
